# baseline (speedup 1.0000x reference)
import functools

import jax
import jax.numpy as jnp
from jax import lax
from jax.experimental import pallas as pl
from jax.experimental.pallas import tpu as pltpu
from jax.experimental.pallas import tpu_sc as plsc

D = 64
B_TOTAL = 16384 * 50

_info = plsc.get_sparse_core_info()
_NC, _NS = _info.num_cores, _info.num_subcores
NW = _NC * _NS
PER_W = B_TOTAL // NW
CHUNK = 512
N_CHUNKS = PER_W // CHUNK


def _make_kernel():
  mesh = plsc.VectorSubcoreMesh(core_axis_name="c", subcore_axis_name="s")

  @functools.partial(
      pl.kernel,
      mesh=mesh,
      out_type=jax.ShapeDtypeStruct((B_TOTAL, D), jnp.float32),
      scratch_types=[
          pltpu.VMEM((PER_W,), jnp.int32),
          pltpu.VMEM((2, CHUNK, D), jnp.float32),
          pltpu.SemaphoreType.DMA,
          pltpu.SemaphoreType.DMA,
      ],
      compiler_params=pltpu.CompilerParams(use_tc_tiling_on_sc=False),
  )
  def emb(idx_hbm, table_hbm, out_hbm, idx_v, rows_v, s0, s1):
    cid = lax.axis_index("c")
    wid = lax.axis_index("s")
    w_base = wid * (2 * PER_W)

    sem_s = (s0, s1)

    def gath_desc(i, b):
      return pltpu.make_async_copy(
          table_hbm.at[idx_v.at[pl.ds((i % N_CHUNKS) * CHUNK, CHUNK)]],
          rows_v.at[b],
          sem_s[b],
      )

    @pl.when(cid == 0)
    def _():
      pltpu.sync_copy(idx_hbm.at[pl.ds(w_base, PER_W)], idx_v)

      def pair(g, carry):
        i = 2 * g
        gath_desc(i, 0).start()
        gath_desc(i + 1, 1).start()
        gath_desc(i, 0).wait()
        gath_desc(i + 1, 1).wait()
        return carry

      lax.fori_loop(0, N_CHUNKS, pair, 0)

  return emb


_emb = _make_kernel()


@jax.jit
def kernel(token_ids, weight):
  idx = token_ids.reshape(-1).astype(jnp.int32) % B_TOTAL
  out = _emb(idx, weight)
  return out.reshape(token_ids.shape[0], token_ids.shape[1], D)


# speedup vs baseline: 1.0197x; 1.0197x over previous
import functools

import jax
import jax.numpy as jnp
from jax import lax
from jax.experimental import pallas as pl
from jax.experimental.pallas import tpu as pltpu
from jax.experimental.pallas import tpu_sc as plsc

D = 64
B_TOTAL = 16384 * 50

_info = plsc.get_sparse_core_info()
_NC, _NS = _info.num_cores, _info.num_subcores
NW = _NC * _NS
PER_W = B_TOTAL // NW
CHUNK = 256
NB = 4
N_CHUNKS = PER_W // CHUNK
assert N_CHUNKS % NB == 0 and N_CHUNKS >= 2 * NB


def _make_kernel():
  mesh = plsc.VectorSubcoreMesh(core_axis_name="c", subcore_axis_name="s")

  @functools.partial(
      pl.kernel,
      mesh=mesh,
      out_type=jax.ShapeDtypeStruct((B_TOTAL, D), jnp.float32),
      scratch_types=[
          pltpu.VMEM((PER_W,), jnp.int32),
          pltpu.VMEM((NB, CHUNK, D), jnp.float32),
          pltpu.SemaphoreType.DMA,
          pltpu.SemaphoreType.DMA,
          pltpu.SemaphoreType.DMA,
          pltpu.SemaphoreType.DMA,
          pltpu.SemaphoreType.DMA,
          pltpu.SemaphoreType.DMA,
          pltpu.SemaphoreType.DMA,
          pltpu.SemaphoreType.DMA,
      ],
      compiler_params=pltpu.CompilerParams(use_tc_tiling_on_sc=False),
  )
  def emb(idx_hbm, table_hbm, out_hbm, idx_v, rows_v,
          g0, g1, g2, g3, s0, s1, s2, s3):
    wid = lax.axis_index("s") * _NC + lax.axis_index("c")
    w_base = wid * PER_W
    pltpu.sync_copy(idx_hbm.at[pl.ds(w_base, PER_W)], idx_v)

    sem_g = (g0, g1, g2, g3)
    sem_s = (s0, s1, s2, s3)

    def gather(i, b):
      return pltpu.make_async_copy(
          table_hbm.at[idx_v.at[pl.ds(i * CHUNK, CHUNK)]],
          rows_v.at[b],
          sem_g[b],
      )

    def store(i, b):
      return pltpu.make_async_copy(
          rows_v.at[b],
          out_hbm.at[pl.ds(w_base + i * CHUNK, CHUNK)],
          sem_s[b],
      )

    gather(0, 0).start()
    gather(1, 1).start()
    gather(0, 0).wait()
    store(0, 0).start()
    gather(2, 2).start()
    gather(1, 1).wait()
    store(1, 1).start()
    gather(3, 3).start()

    def group(grp, carry):
      i0 = 2 + NB * grp
      for j in range(NB):
        i = i0 + j
        b = (2 + j) % NB
        gather(i, b).wait()
        store(i, b).start()
        store(i + 2 - NB, (b + 2) % NB).wait()
        gather(i + 2, (b + 2) % NB).start()
      return carry

    lax.fori_loop(0, (N_CHUNKS - NB) // NB, group, 0)

    for i in range(N_CHUNKS - NB + 2, N_CHUNKS):
      b = i % NB
      gather(i, b).wait()
      store(i, b).start()
    for i in range(N_CHUNKS - NB, N_CHUNKS):
      store(i, i % NB).wait()

  return emb


_emb = _make_kernel()


@jax.jit
def kernel(token_ids, weight):
  idx = token_ids.reshape(-1).astype(jnp.int32)
  out = _emb(idx, weight)
  return out.reshape(token_ids.shape[0], token_ids.shape[1], D)


# speedup vs baseline: 1.0199x; 1.0002x over previous
import functools

import jax
import jax.numpy as jnp
from jax import lax
from jax.experimental import pallas as pl
from jax.experimental.pallas import tpu as pltpu
from jax.experimental.pallas import tpu_sc as plsc

D = 64
B_TOTAL = 16384 * 50

_info = plsc.get_sparse_core_info()
_NC, _NS = _info.num_cores, _info.num_subcores
NW = _NC * _NS
PER_W = B_TOTAL // NW
CHUNK = 320
NB = 4
N_CHUNKS = PER_W // CHUNK
assert N_CHUNKS % NB == 0 and N_CHUNKS >= 2 * NB


def _make_kernel():
  mesh = plsc.VectorSubcoreMesh(core_axis_name="c", subcore_axis_name="s")

  @functools.partial(
      pl.kernel,
      mesh=mesh,
      out_type=jax.ShapeDtypeStruct((B_TOTAL, D), jnp.float32),
      scratch_types=[
          pltpu.VMEM((PER_W,), jnp.int32),
          pltpu.VMEM((NB, CHUNK, D), jnp.float32),
          pltpu.SemaphoreType.DMA,
          pltpu.SemaphoreType.DMA,
          pltpu.SemaphoreType.DMA,
          pltpu.SemaphoreType.DMA,
          pltpu.SemaphoreType.DMA,
          pltpu.SemaphoreType.DMA,
          pltpu.SemaphoreType.DMA,
          pltpu.SemaphoreType.DMA,
      ],
      compiler_params=pltpu.CompilerParams(use_tc_tiling_on_sc=False),
  )
  def emb(idx_hbm, table_hbm, out_hbm, idx_v, rows_v,
          g0, g1, g2, g3, s0, s1, s2, s3):
    wid = lax.axis_index("s") * _NC + lax.axis_index("c")
    w_base = wid * PER_W
    pltpu.sync_copy(idx_hbm.at[pl.ds(w_base, PER_W)], idx_v)

    sem_g = (g0, g1, g2, g3)
    sem_s = (s0, s1, s2, s3)

    def gather(i, b):
      return pltpu.make_async_copy(
          table_hbm.at[idx_v.at[pl.ds(i * CHUNK, CHUNK)]],
          rows_v.at[b],
          sem_g[b],
      )

    def store(i, b):
      return pltpu.make_async_copy(
          rows_v.at[b],
          out_hbm.at[pl.ds(w_base + i * CHUNK, CHUNK)],
          sem_s[b],
      )

    gather(0, 0).start()
    gather(1, 1).start()
    gather(0, 0).wait()
    store(0, 0).start()
    gather(2, 2).start()
    gather(1, 1).wait()
    store(1, 1).start()
    gather(3, 3).start()

    def group(grp, carry):
      i0 = 2 + NB * grp
      for j in range(NB):
        i = i0 + j
        b = (2 + j) % NB
        gather(i, b).wait()
        store(i, b).start()
        store(i + 2 - NB, (b + 2) % NB).wait()
        gather(i + 2, (b + 2) % NB).start()
      return carry

    lax.fori_loop(0, (N_CHUNKS - NB) // NB, group, 0)

    for i in range(N_CHUNKS - NB + 2, N_CHUNKS):
      b = i % NB
      gather(i, b).wait()
      store(i, b).start()
    for i in range(N_CHUNKS - NB, N_CHUNKS):
      store(i, i % NB).wait()

  return emb


_emb = _make_kernel()


@jax.jit
def kernel(token_ids, weight):
  idx = token_ids.reshape(-1).astype(jnp.int32)
  out = _emb(idx, weight)
  return out.reshape(token_ids.shape[0], token_ids.shape[1], D)
